# baseline (device time: 35485 ns/iter reference)
import jax
import jax.numpy as jnp
from jax import lax
from jax.experimental import pallas as pl
from jax.experimental.pallas import tpu as pltpu

N_DEV = 8


def kernel(x, Wg, Wu, Wd):
    m, d = x.shape
    B = m // N_DEV

    def body(x_ref, wg_ref, wu_ref, wd_ref, out_ref,
             rs_send, rs_recv, ag_send, ag_recv, wg_bf, wu_bf, wd_bf,
             rs_send_sems, rs_recv_sems, ag_send_sems, ag_recv_sems):
        my = lax.axis_index("i")
        peers = [lax.rem(my + k, N_DEV) for k in range(1, N_DEV)]

        barrier_sem = pltpu.get_barrier_semaphore()
        for p in peers:
            pl.semaphore_signal(
                barrier_sem, inc=1,
                device_id=(p,), device_id_type=pl.DeviceIdType.MESH,
            )
        pl.semaphore_wait(barrier_sem, N_DEV - 1)

        def rs_desc(p):
            return pltpu.make_async_remote_copy(
                src_ref=rs_send.at[pl.ds(p * B, B), :],
                dst_ref=rs_recv.at[my],
                send_sem=rs_send_sems.at[p],
                recv_sem=rs_recv_sems.at[my],
                device_id=(p,),
                device_id_type=pl.DeviceIdType.MESH,
            )

        def rs_wait_desc(q):
            return pltpu.make_async_remote_copy(
                src_ref=rs_recv.at[q],
                dst_ref=rs_recv.at[q],
                send_sem=rs_recv_sems.at[q],
                recv_sem=rs_recv_sems.at[q],
                device_id=(q,),
                device_id_type=pl.DeviceIdType.MESH,
            )

        wg_bf[:, :] = wg_ref[:, :].astype(jnp.bfloat16)
        wu_bf[:, :] = wu_ref[:, :].astype(jnp.bfloat16)
        wd_bf[:, :] = wd_ref[:, :].astype(jnp.bfloat16)

        R = 2 * B
        NC = m // R
        my_chunk = my >> 1
        for j in range(1, NC + 1):
            c = lax.rem(my_chunk + j, NC)
            rows = pl.ds(c * R, R)
            xs = x_ref[rows, :].astype(jnp.bfloat16)
            gate = jnp.dot(xs, wg_bf[:, :], preferred_element_type=jnp.float32)
            up = jnp.dot(xs, wu_bf[:, :], preferred_element_type=jnp.float32)
            hidden = (gate * (up * jax.nn.sigmoid(up))).astype(jnp.bfloat16)
            partial_c = jnp.dot(hidden, wd_bf[:, :], preferred_element_type=jnp.float32)
            out_ref[rows, :] = partial_c
            rs_send[rows, :] = partial_c.astype(jnp.bfloat16)
            for t in range(2):
                b = 2 * c + t

                @pl.when(b != my)
                def _():
                    rs_desc(b).start()

        acc = out_ref[pl.ds(my * B, B), :]
        for q in peers:
            rs_wait_desc(q).wait_recv()
            acc = acc + rs_recv[q].astype(jnp.float32)
        out_ref[pl.ds(my * B, B), :] = acc
        ag_send[:, :] = acc.astype(jnp.bfloat16)

        def ag_desc(p):
            return pltpu.make_async_remote_copy(
                src_ref=ag_send,
                dst_ref=ag_recv.at[my],
                send_sem=ag_send_sems.at[p],
                recv_sem=ag_recv_sems.at[my],
                device_id=(p,),
                device_id_type=pl.DeviceIdType.MESH,
            )

        def ag_wait_desc(q):
            return pltpu.make_async_remote_copy(
                src_ref=ag_recv.at[q],
                dst_ref=ag_recv.at[q],
                send_sem=ag_recv_sems.at[q],
                recv_sem=ag_recv_sems.at[q],
                device_id=(q,),
                device_id_type=pl.DeviceIdType.MESH,
            )

        for p in peers:
            ag_desc(p).start()

        for q in peers:
            ag_wait_desc(q).wait_recv()
            out_ref[pl.ds(q * B, B), :] = ag_recv[q].astype(jnp.float32)

        for p in peers:
            rs_desc(p).wait_send()
            ag_desc(p).wait_send()

    return pl.pallas_call(
        body,
        out_shape=jax.ShapeDtypeStruct((m, d), jnp.float32),
        in_specs=[pl.BlockSpec(memory_space=pltpu.VMEM)] * 4,
        out_specs=pl.BlockSpec(memory_space=pltpu.VMEM),
        scratch_shapes=[
            pltpu.VMEM((m, d), jnp.bfloat16),
            pltpu.VMEM((N_DEV, B, d), jnp.bfloat16),
            pltpu.VMEM((B, d), jnp.bfloat16),
            pltpu.VMEM((N_DEV, B, d), jnp.bfloat16),
            pltpu.VMEM(Wg.shape, jnp.bfloat16),
            pltpu.VMEM(Wu.shape, jnp.bfloat16),
            pltpu.VMEM(Wd.shape, jnp.bfloat16),
            pltpu.SemaphoreType.DMA((N_DEV,)),
            pltpu.SemaphoreType.DMA((N_DEV,)),
            pltpu.SemaphoreType.DMA((N_DEV,)),
            pltpu.SemaphoreType.DMA((N_DEV,)),
        ],
        compiler_params=pltpu.CompilerParams(collective_id=0),
    )(x, Wg, Wu, Wd)


# device time: 33316 ns/iter; 1.0651x vs baseline; 1.0651x over previous
import jax
import jax.numpy as jnp
from jax import lax
from jax.experimental import pallas as pl
from jax.experimental.pallas import tpu as pltpu

N_DEV = 8


def kernel(x, Wg, Wu, Wd):
    m, d = x.shape
    B = m // N_DEV

    def body(x_ref, wg_ref, wu_ref, wd_ref, out_ref,
             rs_send, rs_recv, ag_send, ag_recv,
             rs_send_sems, rs_recv_sems, ag_send_sems, ag_recv_sems):
        my = lax.axis_index("i")
        peers = [lax.rem(my + k, N_DEV) for k in range(1, N_DEV)]

        barrier_sem = pltpu.get_barrier_semaphore()
        for p in peers:
            pl.semaphore_signal(
                barrier_sem, inc=1,
                device_id=(p,), device_id_type=pl.DeviceIdType.MESH,
            )
        pl.semaphore_wait(barrier_sem, N_DEV - 1)

        def rs_desc(p):
            return pltpu.make_async_remote_copy(
                src_ref=rs_send.at[pl.ds(p * B, B), :],
                dst_ref=rs_recv.at[my],
                send_sem=rs_send_sems.at[p],
                recv_sem=rs_recv_sems.at[my],
                device_id=(p,),
                device_id_type=pl.DeviceIdType.MESH,
            )

        def rs_wait_desc(q):
            return pltpu.make_async_remote_copy(
                src_ref=rs_recv.at[q],
                dst_ref=rs_recv.at[q],
                send_sem=rs_recv_sems.at[q],
                recv_sem=rs_recv_sems.at[q],
                device_id=(q,),
                device_id_type=pl.DeviceIdType.MESH,
            )

        R = 2 * B
        NC = m // R
        my_chunk = my >> 1
        for j in range(1, NC + 1):
            c = lax.rem(my_chunk + j, NC)
            rows = pl.ds(c * R, R)
            xs = x_ref[rows, :]
            gate = jnp.dot(xs, wg_ref[:, :], preferred_element_type=jnp.float32)
            up = jnp.dot(xs, wu_ref[:, :], preferred_element_type=jnp.float32)
            hidden = gate * (up * jax.nn.sigmoid(up))
            partial_c = jnp.dot(hidden, wd_ref[:, :], preferred_element_type=jnp.float32)
            out_ref[rows, :] = partial_c
            rs_send[rows, :] = partial_c.astype(jnp.bfloat16)
            for t in range(2):
                b = 2 * c + t

                @pl.when(b != my)
                def _():
                    rs_desc(b).start()

        acc = out_ref[pl.ds(my * B, B), :]
        for q in peers:
            rs_wait_desc(q).wait_recv()
            acc = acc + rs_recv[q].astype(jnp.float32)
        out_ref[pl.ds(my * B, B), :] = acc

        s_raw = jnp.maximum(jnp.max(jnp.abs(acc)) / 127.0, 1e-30)
        e_ = jnp.floor(jnp.log2(s_raw))
        d_ = jnp.round((s_raw / jnp.exp2(e_)) * 4096.0) - 4096.0
        s_dec = (4096.0 + d_) / 4096.0 * jnp.exp2(e_)
        q_blk = jnp.clip(jnp.round(acc / s_dec), -127.0, 127.0)
        a_ = jnp.mod(d_, 64.0)
        b_ = jnp.floor(d_ / 64.0)
        row_i = lax.broadcasted_iota(jnp.int32, (B, 128), 0)
        col_i = lax.broadcasted_iota(jnp.int32, (B, 128), 1)
        strip = jnp.where(
            (row_i == 0) & (col_i == 0), e_,
            jnp.where((row_i == 0) & (col_i == 1), a_,
                      jnp.where((row_i == 0) & (col_i == 2), b_, 0.0)),
        )
        ag_send[:, :] = jnp.concatenate([q_blk, strip], axis=1).astype(jnp.int8)

        def ag_desc(p):
            return pltpu.make_async_remote_copy(
                src_ref=ag_send,
                dst_ref=ag_recv.at[my],
                send_sem=ag_send_sems.at[p],
                recv_sem=ag_recv_sems.at[my],
                device_id=(p,),
                device_id_type=pl.DeviceIdType.MESH,
            )

        def ag_wait_desc(q):
            return pltpu.make_async_remote_copy(
                src_ref=ag_recv.at[q],
                dst_ref=ag_recv.at[q],
                send_sem=ag_recv_sems.at[q],
                recv_sem=ag_recv_sems.at[q],
                device_id=(q,),
                device_id_type=pl.DeviceIdType.MESH,
            )

        for p in peers:
            ag_desc(p).start()

        for q in peers:
            ag_wait_desc(q).wait_recv()
            blk = ag_recv[q].astype(jnp.float32)
            eab = blk[0, d:d + 3]
            s_q = (4096.0 + eab[1] + 64.0 * eab[2]) / 4096.0 * jnp.exp2(eab[0])
            out_ref[pl.ds(q * B, B), :] = blk[:, :d] * s_q

        for p in peers:
            rs_desc(p).wait_send()
            ag_desc(p).wait_send()

    return pl.pallas_call(
        body,
        out_shape=jax.ShapeDtypeStruct((m, d), jnp.float32),
        in_specs=[pl.BlockSpec(memory_space=pltpu.VMEM)] * 4,
        out_specs=pl.BlockSpec(memory_space=pltpu.VMEM),
        scratch_shapes=[
            pltpu.VMEM((m, d), jnp.bfloat16),
            pltpu.VMEM((N_DEV, B, d), jnp.bfloat16),
            pltpu.VMEM((B, d + 128), jnp.int8),
            pltpu.VMEM((N_DEV, B, d + 128), jnp.int8),
            pltpu.SemaphoreType.DMA((N_DEV,)),
            pltpu.SemaphoreType.DMA((N_DEV,)),
            pltpu.SemaphoreType.DMA((N_DEV,)),
            pltpu.SemaphoreType.DMA((N_DEV,)),
        ],
        compiler_params=pltpu.CompilerParams(collective_id=0),
    )(x, Wg, Wu, Wd)


# device time: 33282 ns/iter; 1.0662x vs baseline; 1.0010x over previous
import jax
import jax.numpy as jnp
from jax import lax
from jax.experimental import pallas as pl
from jax.experimental.pallas import tpu as pltpu

N_DEV = 8


def kernel(x, Wg, Wu, Wd):
    m, d = x.shape
    B = m // N_DEV

    def body(x_ref, wg_ref, wu_ref, wd_ref, out_ref,
             rs_send, rs_recv, ag_send, ag_recv,
             rs_send_sems, rs_recv_sems, ag_send_sems, ag_recv_sems):
        my = lax.axis_index("i")
        peers = [lax.rem(my + k, N_DEV) for k in range(1, N_DEV)]

        barrier_sem = pltpu.get_barrier_semaphore()
        for p in peers:
            pl.semaphore_signal(
                barrier_sem, inc=1,
                device_id=(p,), device_id_type=pl.DeviceIdType.MESH,
            )
        pl.semaphore_wait(barrier_sem, N_DEV - 1)

        def rs_desc(p):
            return pltpu.make_async_remote_copy(
                src_ref=rs_send.at[pl.ds(p * B, B), :],
                dst_ref=rs_recv.at[my],
                send_sem=rs_send_sems.at[p],
                recv_sem=rs_recv_sems.at[my],
                device_id=(p,),
                device_id_type=pl.DeviceIdType.MESH,
            )

        def rs_wait_desc(q):
            return pltpu.make_async_remote_copy(
                src_ref=rs_recv.at[q],
                dst_ref=rs_recv.at[q],
                send_sem=rs_recv_sems.at[q],
                recv_sem=rs_recv_sems.at[q],
                device_id=(q,),
                device_id_type=pl.DeviceIdType.MESH,
            )

        R = 2 * B
        NC = m // R
        my_chunk = my >> 1
        for j in range(1, NC + 1):
            c = lax.rem(my_chunk + j, NC)
            rows = pl.ds(c * R, R)
            xs = x_ref[rows, :]
            gate = jnp.dot(xs, wg_ref[:, :], preferred_element_type=jnp.float32)
            up = jnp.dot(xs, wu_ref[:, :], preferred_element_type=jnp.float32)
            hidden = gate * (up * jax.nn.sigmoid(up))
            partial_c = jnp.dot(hidden, wd_ref[:, :], preferred_element_type=jnp.float32)
            rs_send[rows, :] = partial_c.astype(jnp.bfloat16)
            if j == NC:
                out_ref[rows, :] = partial_c
            for t in range(2):
                b = 2 * c + t

                @pl.when(b != my)
                def _():
                    rs_desc(b).start()

        acc = out_ref[pl.ds(my * B, B), :]
        for q in peers:
            rs_wait_desc(q).wait_recv()
            acc = acc + rs_recv[q].astype(jnp.float32)
        out_ref[pl.ds(my * B, B), :] = acc

        s_raw = jnp.maximum(jnp.max(jnp.abs(acc)) / 127.0, 1e-30)
        e_ = jnp.floor(jnp.log2(s_raw))
        d_ = jnp.round((s_raw / jnp.exp2(e_)) * 4096.0) - 4096.0
        s_dec = (4096.0 + d_) / 4096.0 * jnp.exp2(e_)
        q_blk = jnp.clip(jnp.round(acc / s_dec), -127.0, 127.0)
        a_ = jnp.mod(d_, 64.0)
        b_ = jnp.floor(d_ / 64.0)
        row_i = lax.broadcasted_iota(jnp.int32, (B, 128), 0)
        col_i = lax.broadcasted_iota(jnp.int32, (B, 128), 1)
        strip = jnp.where(
            (row_i == 0) & (col_i == 0), e_,
            jnp.where((row_i == 0) & (col_i == 1), a_,
                      jnp.where((row_i == 0) & (col_i == 2), b_, 0.0)),
        )
        ag_send[:, :] = jnp.concatenate([q_blk, strip], axis=1).astype(jnp.int8)

        def ag_desc(p):
            return pltpu.make_async_remote_copy(
                src_ref=ag_send,
                dst_ref=ag_recv.at[my],
                send_sem=ag_send_sems.at[p],
                recv_sem=ag_recv_sems.at[my],
                device_id=(p,),
                device_id_type=pl.DeviceIdType.MESH,
            )

        def ag_wait_desc(q):
            return pltpu.make_async_remote_copy(
                src_ref=ag_recv.at[q],
                dst_ref=ag_recv.at[q],
                send_sem=ag_recv_sems.at[q],
                recv_sem=ag_recv_sems.at[q],
                device_id=(q,),
                device_id_type=pl.DeviceIdType.MESH,
            )

        for p in peers:
            ag_desc(p).start()

        for q in peers:
            ag_wait_desc(q).wait_recv()
            blk = ag_recv[q].astype(jnp.float32)
            eab = blk[0, d:d + 3]
            s_q = (4096.0 + eab[1] + 64.0 * eab[2]) / 4096.0 * jnp.exp2(eab[0])
            out_ref[pl.ds(q * B, B), :] = blk[:, :d] * s_q

        for p in peers:
            rs_desc(p).wait_send()
            ag_desc(p).wait_send()

    return pl.pallas_call(
        body,
        out_shape=jax.ShapeDtypeStruct((m, d), jnp.float32),
        in_specs=[pl.BlockSpec(memory_space=pltpu.VMEM)] * 4,
        out_specs=pl.BlockSpec(memory_space=pltpu.VMEM),
        scratch_shapes=[
            pltpu.VMEM((m, d), jnp.bfloat16),
            pltpu.VMEM((N_DEV, B, d), jnp.bfloat16),
            pltpu.VMEM((B, d + 128), jnp.int8),
            pltpu.VMEM((N_DEV, B, d + 128), jnp.int8),
            pltpu.SemaphoreType.DMA((N_DEV,)),
            pltpu.SemaphoreType.DMA((N_DEV,)),
            pltpu.SemaphoreType.DMA((N_DEV,)),
            pltpu.SemaphoreType.DMA((N_DEV,)),
        ],
        compiler_params=pltpu.CompilerParams(collective_id=0),
    )(x, Wg, Wu, Wd)
